# Initial kernel scaffold; baseline (speedup 1.0000x reference)
import jax
import jax.numpy as jnp
from jax.experimental import pallas as pl


def kernel(x, Wr, W1, b1, W2, b2):
    raise NotImplementedError("write your pallas kernel here")


# speedup vs baseline: 1.5182x; 1.5182x over previous
import jax
import jax.numpy as jnp
from jax.experimental import pallas as pl
from jax.experimental.pallas import tpu as pltpu

T = 2048
D = 1024
F = 2048
E = 8
K = 2

TB = 512


def _moe_dense_kernel(x_ref, wr_ref, w1_ref, b1_ref, w2_ref, b2_ref, out_ref):
    e = pl.program_id(0)
    t = pl.program_id(1)
    xb = x_ref[...]

    logits = jnp.dot(xb, wr_ref[...], preferred_element_type=jnp.float32)
    p = jax.nn.softmax(logits, axis=-1)
    ii = jax.lax.broadcasted_iota(jnp.int32, p.shape, 1)
    m1 = jnp.max(p, axis=-1, keepdims=True)
    i1 = jnp.min(jnp.where(p >= m1, ii, E), axis=-1, keepdims=True)
    p2 = jnp.where(ii == i1, -jnp.inf, p)
    m2 = jnp.max(p2, axis=-1, keepdims=True)
    i2 = jnp.min(jnp.where(p2 >= m2, ii, E), axis=-1, keepdims=True)
    combine = (jnp.where(ii == i1, m1, 0.0) + jnp.where(ii == i2, m2, 0.0))

    h = jnp.dot(xb, w1_ref[0], preferred_element_type=jnp.float32) + b1_ref[0]
    h = jax.nn.gelu(h)
    y = jnp.dot(h, w2_ref[0], preferred_element_type=jnp.float32) + b2_ref[0]
    ce = jnp.sum(jnp.where(ii == e, combine, 0.0), axis=-1, keepdims=True)
    contrib = ce * y

    rows = pl.ds(t * TB, TB)

    @pl.when(e == 0)
    def _():
        out_ref[rows, :] = contrib

    @pl.when(e != 0)
    def _():
        out_ref[rows, :] += contrib


def kernel(x, Wr, W1, b1, W2, b2):
    grid = (E, T // TB)
    b1r = b1.reshape(E, 1, F)
    b2r = b2.reshape(E, 1, D)
    return pl.pallas_call(
        _moe_dense_kernel,
        grid=grid,
        in_specs=[
            pl.BlockSpec((TB, D), lambda e, t: (t, 0)),
            pl.BlockSpec((D, E), lambda e, t: (0, 0)),
            pl.BlockSpec((1, D, F), lambda e, t: (e, 0, 0)),
            pl.BlockSpec((1, 1, F), lambda e, t: (e, 0, 0)),
            pl.BlockSpec((1, F, D), lambda e, t: (e, 0, 0)),
            pl.BlockSpec((1, 1, D), lambda e, t: (e, 0, 0)),
        ],
        out_specs=pl.BlockSpec((T, D), lambda e, t: (0, 0)),
        out_shape=jax.ShapeDtypeStruct((T, D), jnp.float32),
        compiler_params=pltpu.CompilerParams(
            dimension_semantics=("arbitrary", "arbitrary"),
        ),
    )(x, Wr, W1, b1r, W2, b2r)
